# baseline (speedup 1.0000x reference)
import functools
import math

import jax
import jax.numpy as jnp
import numpy as np
from jax import lax
from jax.experimental import pallas as pl
from jax.experimental.pallas import tpu as pltpu
from jax.experimental.pallas import tpu_sc as plsc

VOCAB = 100000
D = 128
SCALE = math.sqrt(D)

_info = plsc.get_sparse_core_info()
NC, NS = _info.num_cores, _info.num_subcores
NW = NC * NS

B_TOTAL = 4096 * 200
B_PER_W = B_TOTAL // NW
CHUNK = 200
N_CHUNKS = B_PER_W // CHUNK


_PERM = np.zeros((D, D), np.float32)
for _g in range(4):
    for _l in range(16):
        _PERM[_g * 32 + _l, _g * 32 + 2 * _l] = 1.0
        _PERM[_g * 32 + 16 + _l, _g * 32 + 2 * _l + 1] = 1.0


def _prescale_body(t_ref, p_ref, o_ref):
    y = jnp.dot(t_ref[...], p_ref[...], preferred_element_type=jnp.float32)
    o_ref[...] = (y * SCALE).astype(jnp.bfloat16)


def _prescale(table):
    rows_per_blk = 4000
    return pl.pallas_call(
        _prescale_body,
        out_shape=jax.ShapeDtypeStruct((VOCAB, D), jnp.bfloat16),
        grid=(VOCAB // rows_per_blk,),
        in_specs=[pl.BlockSpec((rows_per_blk, D), lambda i: (i, 0)),
                  pl.BlockSpec((D, D), lambda i: (0, 0))],
        out_specs=pl.BlockSpec((rows_per_blk, D), lambda i: (i, 0)),
    )(table, jnp.asarray(_PERM))


def _gather_body(table_hbm, idx_hbm, out_hbm, idx_v,
                 bf0, bf1, f0, f1, g0, g1, o0, o1):
    wid = lax.axis_index("s") * NC + lax.axis_index("c")
    base = wid * B_PER_W
    bfbufs = (bf0, bf1)
    fbufs = (f0, f1)
    gsems = (g0, g1)
    osems = (o0, o1)

    pltpu.sync_copy(idx_hbm.at[pl.ds(base, B_PER_W)], idx_v)


    def g_start(i, b):
        pltpu.async_copy(table_hbm.at[idx_v.at[pl.ds(i * CHUNK, CHUNK)]],
                         bfbufs[b], gsems[b])

    def g_wait(b):
        pltpu.make_async_copy(table_hbm.at[idx_v.at[pl.ds(0, CHUNK)]],
                              bfbufs[b], gsems[b]).wait()

    def o_start(i, b):
        pltpu.async_copy(fbufs[b], out_hbm.at[pl.ds(base + i * CHUNK, CHUNK)],
                         osems[b])

    def o_wait(b):
        pltpu.make_async_copy(fbufs[b], out_hbm.at[pl.ds(base, CHUNK)],
                              osems[b]).wait()

    def convert(b):
        bfb, fb = bfbufs[b], fbufs[b]

        def rows(k, carry):
            r = k * 2
            for dr in range(2):
                for g in range(4):
                    w = bfb[r + dr, pl.ds(g * 16, 16)]
                    lo = lax.bitcast_convert_type(w << 16, jnp.float32)
                    hi = lax.bitcast_convert_type(w & jnp.int32(-65536),
                                                  jnp.float32)
                    fb[r + dr, pl.ds(g * 32, 16)] = lo
                    fb[r + dr, pl.ds(g * 32 + 16, 16)] = hi
            return carry

        lax.fori_loop(0, CHUNK // 2, rows, 0)

    g_start(0, 0)
    g_start(1, 1)

    def outer(k, carry):
        i0 = k * 2
        for b in range(2):
            i = i0 + b

            @pl.when(i >= 2)
            def _():
                o_wait(b)

            g_wait(b)
            convert(b)
            o_start(i, b)

            @pl.when(i + 2 < N_CHUNKS)
            def _():
                g_start(i + 2, b)
        return carry

    lax.fori_loop(0, N_CHUNKS // 2, outer, 0)
    o_wait(0)
    o_wait(1)


_gather = functools.partial(
    pl.kernel,
    mesh=plsc.VectorSubcoreMesh(core_axis_name="c", subcore_axis_name="s"),
    compiler_params=pltpu.CompilerParams(use_tc_tiling_on_sc=False),
    out_type=jax.ShapeDtypeStruct((B_TOTAL, D), jnp.float32),
    scratch_types=[
        pltpu.VMEM((B_PER_W,), jnp.int32),
        pltpu.VMEM((CHUNK, D // 2), jnp.int32),
        pltpu.VMEM((CHUNK, D // 2), jnp.int32),
        pltpu.VMEM((CHUNK, D), jnp.float32),
        pltpu.VMEM((CHUNK, D), jnp.float32),
        pltpu.SemaphoreType.DMA,
        pltpu.SemaphoreType.DMA,
        pltpu.SemaphoreType.DMA,
        pltpu.SemaphoreType.DMA,
    ],
)(_gather_body)


def kernel(x, table):
    idx = x.reshape(-1).astype(jnp.int32)
    scaled = _prescale(table)
    scaled_w = lax.bitcast_convert_type(
        scaled.reshape(VOCAB, D // 2, 2), jnp.int32)
    out = _gather(scaled_w, idx)
    return out.reshape(x.shape + (D,))


# speedup vs baseline: 3.1533x; 3.1533x over previous
import functools
import math

import jax
import jax.numpy as jnp
from jax import lax
from jax.experimental import pallas as pl
from jax.experimental.pallas import tpu as pltpu
from jax.experimental.pallas import tpu_sc as plsc

VOCAB = 100000
D = 128
SCALE = math.sqrt(D)

_info = plsc.get_sparse_core_info()
NC, NS = _info.num_cores, _info.num_subcores
NW = NC * NS

B_TOTAL = 4096 * 200
B_PER_W = B_TOTAL // NW
CHUNK = 200
N_CHUNKS = B_PER_W // CHUNK
NBUF = 4


def _gather_body(table_hbm, idx_hbm, out_hbm, idx_v,
                 buf0, buf1, buf2, buf3,
                 g0, g1, g2, g3, o0, o1, o2, o3):
    wid = lax.axis_index("s") * NC + lax.axis_index("c")
    base = wid * B_PER_W
    bufs = (buf0, buf1, buf2, buf3)
    gsems = (g0, g1, g2, g3)
    osems = (o0, o1, o2, o3)

    pltpu.sync_copy(idx_hbm.at[pl.ds(base, B_PER_W)], idx_v)

    def g_start(i, b):
        pltpu.async_copy(table_hbm.at[idx_v.at[pl.ds(i * CHUNK, CHUNK)]],
                         bufs[b], gsems[b])

    def g_wait(b):
        pltpu.make_async_copy(table_hbm.at[idx_v.at[pl.ds(0, CHUNK)]],
                              bufs[b], gsems[b]).wait()

    def o_start(i, b):
        pltpu.async_copy(bufs[b], out_hbm.at[pl.ds(base + i * CHUNK, CHUNK)],
                         osems[b])

    def o_wait(b):
        pltpu.make_async_copy(bufs[b], out_hbm.at[pl.ds(base, CHUNK)],
                              osems[b]).wait()

    def scale(buf):
        def rows(k, carry):
            r = k * 4
            for dr in range(4):
                for j in range(8):
                    sl = (r + dr, pl.ds(j * 16, 16))
                    buf[sl] = buf[sl] * SCALE
            return carry

        lax.fori_loop(0, CHUNK // 4, rows, 0)

    g_start(0, 0)
    g_start(1, 1)

    def outer(k, carry):
        i0 = k * NBUF
        for b in range(NBUF):
            i = i0 + b
            b2 = (b + 2) % NBUF
            g_wait(b)

            @pl.when(i + 2 < N_CHUNKS)
            def _():
                @pl.when(i >= 2)
                def _():
                    o_wait(b2)

                g_start(i + 2, b2)

            scale(bufs[b])
            o_start(i, b)
        return carry

    lax.fori_loop(0, N_CHUNKS // NBUF, outer, 0)
    for b in range(NBUF):
        o_wait(b)


_gather = functools.partial(
    pl.kernel,
    mesh=plsc.VectorSubcoreMesh(core_axis_name="c", subcore_axis_name="s"),
    out_type=jax.ShapeDtypeStruct((B_TOTAL, D), jnp.float32),
    scratch_types=[
        pltpu.VMEM((B_PER_W,), jnp.int32),
        pltpu.VMEM((CHUNK, D), jnp.float32),
        pltpu.VMEM((CHUNK, D), jnp.float32),
        pltpu.VMEM((CHUNK, D), jnp.float32),
        pltpu.VMEM((CHUNK, D), jnp.float32),
        pltpu.SemaphoreType.DMA,
        pltpu.SemaphoreType.DMA,
        pltpu.SemaphoreType.DMA,
        pltpu.SemaphoreType.DMA,
        pltpu.SemaphoreType.DMA,
        pltpu.SemaphoreType.DMA,
        pltpu.SemaphoreType.DMA,
        pltpu.SemaphoreType.DMA,
    ],
)(_gather_body)


def kernel(x, table):
    idx = x.reshape(-1).astype(jnp.int32)
    out = _gather(table, idx)
    return out.reshape(x.shape + (D,))


# speedup vs baseline: 3.1548x; 1.0005x over previous
import functools
import math

import jax
import jax.numpy as jnp
from jax import lax
from jax.experimental import pallas as pl
from jax.experimental.pallas import tpu as pltpu
from jax.experimental.pallas import tpu_sc as plsc

VOCAB = 100000
D = 128
SCALE = math.sqrt(D)

_info = plsc.get_sparse_core_info()
NC, NS = _info.num_cores, _info.num_subcores
NW = NC * NS

B_TOTAL = 4096 * 200
B_PER_W = B_TOTAL // NW
CHUNK = 160
N_CHUNKS = B_PER_W // CHUNK
NBUF = 5


def _gather_body(table_hbm, idx_hbm, out_hbm, idx_v,
                 buf0, buf1, buf2, buf3, buf4,
                 g0, g1, g2, g3, g4, o0, o1, o2, o3, o4):
    wid = lax.axis_index("s") * NC + lax.axis_index("c")
    base = wid * B_PER_W
    bufs = (buf0, buf1, buf2, buf3, buf4)
    gsems = (g0, g1, g2, g3, g4)
    osems = (o0, o1, o2, o3, o4)

    pltpu.sync_copy(idx_hbm.at[pl.ds(base, B_PER_W)], idx_v)

    def g_start(i, b):
        pltpu.async_copy(table_hbm.at[idx_v.at[pl.ds(i * CHUNK, CHUNK)]],
                         bufs[b], gsems[b])

    def g_wait(b):
        pltpu.make_async_copy(table_hbm.at[idx_v.at[pl.ds(0, CHUNK)]],
                              bufs[b], gsems[b]).wait()

    def o_start(i, b):
        pltpu.async_copy(bufs[b], out_hbm.at[pl.ds(base + i * CHUNK, CHUNK)],
                         osems[b])

    def o_wait(b):
        pltpu.make_async_copy(bufs[b], out_hbm.at[pl.ds(base, CHUNK)],
                              osems[b]).wait()

    def scale(buf):
        def rows(k, carry):
            r = k * 4
            for dr in range(4):
                for j in range(8):
                    sl = (r + dr, pl.ds(j * 16, 16))
                    buf[sl] = buf[sl] * SCALE
            return carry

        lax.fori_loop(0, CHUNK // 4, rows, 0)

    g_start(0, 0)
    g_start(1, 1)
    g_start(2, 2)

    def outer(k, carry):
        i0 = k * NBUF
        for b in range(NBUF):
            i = i0 + b
            b3 = (b + 3) % NBUF
            g_wait(b)

            @pl.when(i + 3 < N_CHUNKS)
            def _():
                @pl.when(i >= 2)
                def _():
                    o_wait(b3)

                g_start(i + 3, b3)

            scale(bufs[b])
            o_start(i, b)
        return carry

    lax.fori_loop(0, N_CHUNKS // NBUF, outer, 0)
    for b in range(NBUF):
        o_wait(b)


_gather = functools.partial(
    pl.kernel,
    mesh=plsc.VectorSubcoreMesh(core_axis_name="c", subcore_axis_name="s"),
    out_type=jax.ShapeDtypeStruct((B_TOTAL, D), jnp.float32),
    scratch_types=[
        pltpu.VMEM((B_PER_W,), jnp.int32),
        pltpu.VMEM((CHUNK, D), jnp.float32),
        pltpu.VMEM((CHUNK, D), jnp.float32),
        pltpu.VMEM((CHUNK, D), jnp.float32),
        pltpu.VMEM((CHUNK, D), jnp.float32),
        pltpu.VMEM((CHUNK, D), jnp.float32),
        pltpu.SemaphoreType.DMA,
        pltpu.SemaphoreType.DMA,
        pltpu.SemaphoreType.DMA,
        pltpu.SemaphoreType.DMA,
        pltpu.SemaphoreType.DMA,
        pltpu.SemaphoreType.DMA,
        pltpu.SemaphoreType.DMA,
        pltpu.SemaphoreType.DMA,
        pltpu.SemaphoreType.DMA,
        pltpu.SemaphoreType.DMA,
    ],
)(_gather_body)


def kernel(x, table):
    idx = x.reshape(-1).astype(jnp.int32)
    out = _gather(table, idx)
    return out.reshape(x.shape + (D,))
